# baseline (speedup 1.0000x reference)
import functools

import jax
import jax.numpy as jnp
from jax import lax
from jax.experimental import pallas as pl
from jax.experimental.pallas import tpu as pltpu
from jax.experimental.pallas import tpu_sc as plsc

VOCAB = 1000000
SEQ = 200
DIM = 64
BATCH = 1024

NC = 2
NS = 16
NW = NC * NS
ROWS = BATCH * SEQ
RPW = ROWS // NW
CHUNK = SEQ
NCHUNK = RPW // CHUNK
PCHUNK = CHUNK // 2
FCHUNK = CHUNK * DIM

_mesh = plsc.VectorSubcoreMesh(core_axis_name="c", subcore_axis_name="s")


@functools.partial(
    pl.kernel,
    out_type=jax.ShapeDtypeStruct((ROWS * DIM,), jnp.float32),
    mesh=_mesh,
    compiler_params=pltpu.CompilerParams(needs_layout_passes=False),
    scratch_types=[
        pltpu.VMEM((RPW,), jnp.int32),
        pltpu.VMEM((RPW,), jnp.int32),
        pltpu.VMEM((CHUNK, 2 * DIM), jnp.float32),
        pltpu.VMEM((CHUNK, 2 * DIM), jnp.float32),
        pltpu.VMEM((FCHUNK,), jnp.float32),
        pltpu.VMEM((FCHUNK,), jnp.float32),
        pltpu.VMEM((FCHUNK,), jnp.float32),
        pltpu.SemaphoreType.DMA,
        pltpu.SemaphoreType.DMA,
        pltpu.SemaphoreType.DMA,
        pltpu.SemaphoreType.DMA,
    ],
)
def _embed(tab_hbm, idx_hbm, off_hbm, pos_hbm, out_hbm,
           idx_v, off_v, rows0, rows1, outb0, outb1, pos_v, g0, g1, s0, s1):
    wid = lax.axis_index("s") * NC + lax.axis_index("c")
    base = wid * RPW
    fbase = wid * (RPW * DIM)
    pltpu.sync_copy(idx_hbm.at[pl.ds(base, RPW)], idx_v)
    pltpu.sync_copy(off_hbm.at[pl.ds(base, RPW)], off_v)
    pltpu.sync_copy(pos_hbm, pos_v)

    lanes = lax.iota(jnp.int32, 16)

    def start_gather(ci, rows, sem):
        pltpu.async_copy(
            tab_hbm.at[idx_v.at[pl.ds(ci * CHUNK, CHUNK)]], rows, sem)

    def wait_gather(rows, sem):
        pltpu.make_async_copy(
            tab_hbm.at[idx_v.at[pl.ds(0, CHUNK)]], rows, sem).wait()

    def start_store(ci, outb, sem):
        pltpu.async_copy(
            outb, out_hbm.at[pl.ds(fbase + ci * FCHUNK, FCHUNK)], sem)

    def wait_store(outb, sem):
        pltpu.make_async_copy(
            outb, out_hbm.at[pl.ds(fbase, FCHUNK)], sem).wait()

    def assemble(ci, rows, outb):
        obase = ci * CHUNK

        @plsc.parallel_loop(0, PCHUNK, 1, unroll=2)
        def _(p):
            r0 = 2 * p
            fb = 128 * p
            off0 = plsc.load_gather(off_v, [jnp.full((16,), obase + r0,
                                                     jnp.int32)])
            off1 = plsc.load_gather(off_v, [jnp.full((16,), obase + r0 + 1,
                                                     jnp.int32)])
            row0 = jnp.full((16,), r0, jnp.int32)
            row1 = jnp.full((16,), r0 + 1, jnp.int32)
            for c in range(4):
                cols = lanes + c * 16
                sl = pl.ds(fb + c * 16, 16)
                val = plsc.load_gather(rows, [row0, off0 + cols])
                outb[sl] = val + pos_v[sl]
            for c in range(4):
                cols = lanes + c * 16
                sl = pl.ds(fb + DIM + c * 16, 16)
                val = plsc.load_gather(rows, [row1, off1 + cols])
                outb[sl] = val + pos_v[sl]

    def pair(g, _):
        ci0 = 2 * g
        ci1 = ci0 + 1

        start_gather(ci0, rows0, g0)
        start_gather(ci1, rows1, g1)

        wait_gather(rows0, g0)

        @pl.when(g > 0)
        def _():
            wait_store(outb0, s0)

        assemble(ci0, rows0, outb0)
        start_store(ci0, outb0, s0)

        wait_gather(rows1, g1)

        @pl.when(g > 0)
        def _():
            wait_store(outb1, s1)

        assemble(ci1, rows1, outb1)
        start_store(ci1, outb1, s1)
        return 0

    lax.fori_loop(0, NCHUNK // 2, pair, 0)
    wait_store(outb0, s0)
    wait_store(outb1, s1)


def kernel(x, token_table, pos_table):
    xf = x.reshape(-1).astype(jnp.int32)
    idx2 = xf >> 1
    off = (xf & 1) << 6
    tab2 = token_table.reshape(VOCAB // 2, 2 * DIM)
    posf = pos_table.reshape(-1)
    out = _embed(tab2, idx2, off, posf)
    return out.reshape(BATCH, SEQ, DIM)


# speedup vs baseline: 1.0923x; 1.0923x over previous
import functools

import jax
import jax.numpy as jnp
from jax import lax
from jax.experimental import pallas as pl
from jax.experimental.pallas import tpu as pltpu
from jax.experimental.pallas import tpu_sc as plsc

VOCAB = 1000000
SEQ = 200
DIM = 64
BATCH = 1024

NC = 2
NS = 16
NW = NC * NS
ROWS = BATCH * SEQ
RPW = ROWS // NW
CHUNK = SEQ
NCHUNK = RPW // CHUNK

_mesh = plsc.VectorSubcoreMesh(core_axis_name="c", subcore_axis_name="s")


@functools.partial(
    pl.kernel,
    out_type=jax.ShapeDtypeStruct((BATCH, SEQ, DIM), jnp.float32),
    mesh=_mesh,
    compiler_params=pltpu.CompilerParams(use_tc_tiling_on_sc=False,
                                         needs_layout_passes=False),
    scratch_types=[
        pltpu.VMEM((RPW,), jnp.int32),
        pltpu.VMEM((CHUNK, 2 * DIM), jnp.float32),
        pltpu.VMEM((CHUNK, 2 * DIM), jnp.float32),
        pltpu.VMEM((CHUNK, DIM), jnp.float32),
        pltpu.VMEM((CHUNK, DIM), jnp.float32),
        pltpu.VMEM((SEQ, DIM), jnp.float32),
        pltpu.SemaphoreType.DMA,
        pltpu.SemaphoreType.DMA,
        pltpu.SemaphoreType.DMA,
        pltpu.SemaphoreType.DMA,
    ],
)
def _embed(tab_hbm, idx_hbm, pos_hbm, out_hbm,
           idx_v, rows0, rows1, outb0, outb1, pos_v, g0, g1, s0, s1):
    wid = lax.axis_index("s") * NC + lax.axis_index("c")
    base = wid * RPW
    bbase = wid * NCHUNK
    pltpu.sync_copy(idx_hbm.at[pl.ds(base, RPW)], idx_v)
    pltpu.sync_copy(pos_hbm, pos_v)

    def start_gather(ci, rows, sem):
        pltpu.async_copy(
            tab_hbm.at[idx_v.at[pl.ds(ci * CHUNK, CHUNK)]], rows, sem)

    def wait_gather(rows, sem):
        pltpu.make_async_copy(
            tab_hbm.at[idx_v.at[pl.ds(0, CHUNK)]], rows, sem).wait()

    def start_store(ci, outb, sem):
        pltpu.async_copy(outb, out_hbm.at[bbase + ci], sem)

    def wait_store(outb, sem):
        pltpu.make_async_copy(outb, out_hbm.at[bbase], sem).wait()

    def assemble(rows, outb):
        @plsc.parallel_loop(0, CHUNK, 1, unroll=4)
        def _(r):
            for c in range(DIM // 16):
                sl = pl.ds(c * 16, 16)
                outb[r, sl] = rows[r, sl] + pos_v[r, sl]

    def pair(g, _):
        ci0 = 2 * g
        ci1 = ci0 + 1

        start_gather(ci0, rows0, g0)
        start_gather(ci1, rows1, g1)

        wait_gather(rows0, g0)

        @pl.when(g > 0)
        def _():
            wait_store(outb0, s0)

        assemble(rows0, outb0)
        start_store(ci0, outb0, s0)

        wait_gather(rows1, g1)

        @pl.when(g > 0)
        def _():
            wait_store(outb1, s1)

        assemble(rows1, outb1)
        start_store(ci1, outb1, s1)
        return 0

    lax.fori_loop(0, NCHUNK // 2, pair, 0)
    wait_store(outb0, s0)
    wait_store(outb1, s1)


def kernel(x, token_table, pos_table):
    xf = x.reshape(-1).astype(jnp.int32)
    tabp = jnp.pad(token_table, ((0, 0), (0, DIM)))
    out = _embed(tabp, xf, pos_table)
    return out
